# Initial kernel scaffold; baseline (speedup 1.0000x reference)
import functools

import jax
import jax.numpy as jnp
from jax import lax
from jax.experimental import pallas as pl
from jax.experimental.pallas import tpu as pltpu
from jax.experimental.pallas import tpu_sc as plsc

NC = 2
NS = 16
NW = NC * NS
BLK = 128


def _sc_embedding_lookup(idx2d, weight):
    nblk, blk = idx2d.shape
    n_rows, d = weight.shape
    blks_per_w = nblk // NW
    b_total = nblk * blk

    mesh = plsc.VectorSubcoreMesh(core_axis_name="c", subcore_axis_name="s")

    @functools.partial(
        pl.kernel,
        out_type=jax.ShapeDtypeStruct((b_total, d), jnp.float32),
        mesh=mesh,
        scratch_types=[
            pltpu.VMEM((blks_per_w, blk), jnp.int32),
            pltpu.VMEM((blk, d), jnp.float32),
            pltpu.SemaphoreType.DMA,
        ],
    )
    def k(idx_hbm, w_hbm, out_hbm, idx_v, rows_v, sem):
        wid = lax.axis_index("s") * NC + lax.axis_index("c")
        blk0 = wid * blks_per_w
        pltpu.sync_copy(idx_hbm.at[pl.ds(blk0, blks_per_w)], idx_v)

        def body(j, carry):
            pltpu.async_copy(w_hbm.at[idx_v.at[j]], rows_v, sem).wait()
            pltpu.sync_copy(rows_v, out_hbm.at[pl.ds((blk0 + j) * blk, blk)])
            return carry

        lax.fori_loop(0, blks_per_w, body, 0)

    return k(idx2d, weight)


def kernel(x, weight):
    b0, b1 = x.shape
    d = weight.shape[1]
    idx2d = x.reshape(-1, BLK).astype(jnp.int32)
    out = _sc_embedding_lookup(idx2d, weight)
    return out.reshape(b0, b1, d)


# speedup vs baseline: 1.0235x; 1.0235x over previous
import functools

import jax
import jax.numpy as jnp
from jax import lax
from jax.experimental import pallas as pl
from jax.experimental.pallas import tpu as pltpu
from jax.experimental.pallas import tpu_sc as plsc

NC = 2
NS = 16
NW = NC * NS
BLK = 128


def _sc_embedding_lookup(idx2d, weight):
    nblk, blk = idx2d.shape
    n_rows, d = weight.shape
    blks_per_w = nblk // NW
    b_total = nblk * blk

    mesh = plsc.VectorSubcoreMesh(core_axis_name="c", subcore_axis_name="s")

    @functools.partial(
        pl.kernel,
        out_type=jax.ShapeDtypeStruct((b_total, d), jnp.float32),
        mesh=mesh,
        scratch_types=[
            pltpu.VMEM((blks_per_w, blk), jnp.int32),
            pltpu.VMEM((blk, d), jnp.float32),
            pltpu.SemaphoreType.DMA,
        ],
        compiler_params=pltpu.CompilerParams(use_tc_tiling_on_sc=False),
    )
    def k(idx_hbm, w_hbm, out_hbm, idx_v, rows_v, sem):
        wid = lax.axis_index("s") * NC + lax.axis_index("c")
        blk0 = wid * blks_per_w
        pltpu.sync_copy(idx_hbm.at[pl.ds(blk0, blks_per_w)], idx_v)

        def body(j, carry):
            pltpu.async_copy(w_hbm.at[idx_v.at[j]], rows_v, sem).wait()
            pltpu.sync_copy(rows_v, out_hbm.at[pl.ds((blk0 + j) * blk, blk)])
            return carry

        lax.fori_loop(0, blks_per_w, body, 0)

    return k(idx2d, weight)


def kernel(x, weight):
    b0, b1 = x.shape
    d = weight.shape[1]
    idx2d = x.reshape(-1, BLK).astype(jnp.int32)
    out = _sc_embedding_lookup(idx2d, weight)
    return out.reshape(b0, b1, d)


# speedup vs baseline: 1.1122x; 1.0867x over previous
import functools

import jax
import jax.numpy as jnp
from jax import lax
from jax.experimental import pallas as pl
from jax.experimental.pallas import tpu as pltpu
from jax.experimental.pallas import tpu_sc as plsc

NC = 2
NS = 16
NW = NC * NS
BLK = 128
NBUF = 8


def _sc_embedding_lookup(idx2d, weight):
    nblk, blk = idx2d.shape
    n_rows, d = weight.shape
    blks_per_w = nblk // NW
    b_total = nblk * blk
    ngroups = blks_per_w // NBUF

    mesh = plsc.VectorSubcoreMesh(core_axis_name="c", subcore_axis_name="s")

    @functools.partial(
        pl.kernel,
        out_type=jax.ShapeDtypeStruct((b_total, d), jnp.float32),
        mesh=mesh,
        scratch_types=[
            pltpu.VMEM((blks_per_w, blk), jnp.int32),
            *([pltpu.VMEM((blk, d), jnp.float32)] * NBUF),
            *([pltpu.SemaphoreType.DMA] * NBUF),
            *([pltpu.SemaphoreType.DMA] * NBUF),
        ],
        compiler_params=pltpu.CompilerParams(use_tc_tiling_on_sc=False),
    )
    def k(idx_hbm, w_hbm, out_hbm, idx_v, *bufs_and_sems):
        bufs = bufs_and_sems[:NBUF]
        sem_g = bufs_and_sems[NBUF:2 * NBUF]
        sem_o = bufs_and_sems[2 * NBUF:3 * NBUF]

        wid = lax.axis_index("s") * NC + lax.axis_index("c")
        blk0 = wid * blks_per_w
        pltpu.sync_copy(idx_hbm.at[pl.ds(blk0, blks_per_w)], idx_v)

        def start_gather(j, b):
            pltpu.async_copy(w_hbm.at[idx_v.at[j]], bufs[b], sem_g[b])

        def wait_gather(b):
            pltpu.make_async_copy(w_hbm.at[idx_v.at[0]], bufs[b], sem_g[b]).wait()

        def out_slice(j):
            return out_hbm.at[pl.ds((blk0 + j) * blk, blk)]

        def start_out(j, b):
            pltpu.async_copy(bufs[b], out_slice(j), sem_o[b])

        def wait_out(b):
            pltpu.make_async_copy(bufs[b], out_slice(0), sem_o[b]).wait()

        for b in range(NBUF):
            start_gather(b, b)

        def group_body(g, carry):
            j0 = g * NBUF
            for b in range(NBUF):
                wait_gather(b)
                start_out(j0 + b, b)
            for b in range(NBUF):
                wait_out(b)
                start_gather(j0 + NBUF + b, b)
            return carry

        lax.fori_loop(0, ngroups - 1, group_body, 0)

        j0 = (ngroups - 1) * NBUF
        for b in range(NBUF):
            wait_gather(b)
            start_out(j0 + b, b)
        for b in range(NBUF):
            wait_out(b)

    return k(idx2d, weight)


def kernel(x, weight):
    b0, b1 = x.shape
    d = weight.shape[1]
    idx2d = x.reshape(-1, BLK).astype(jnp.int32)
    out = _sc_embedding_lookup(idx2d, weight)
    return out.reshape(b0, b1, d)


# speedup vs baseline: 1.3055x; 1.1738x over previous
import functools

import jax
import jax.numpy as jnp
from jax import lax
from jax.experimental import pallas as pl
from jax.experimental.pallas import tpu as pltpu
from jax.experimental.pallas import tpu_sc as plsc

NC = 2
NS = 16
NW = NC * NS
BLK = 128
HALF = 5
NBUF = 4


def _sc_embedding_lookup(idx2d, weight):
    nblk, blk = idx2d.shape
    n_rows, d = weight.shape
    blks_per_w = nblk // NW
    nfills = blks_per_w // HALF
    ngroups = nfills // NBUF

    mesh = plsc.VectorSubcoreMesh(core_axis_name="c", subcore_axis_name="s")

    @functools.partial(
        pl.kernel,
        out_type=jax.ShapeDtypeStruct((nblk, blk, d), jnp.float32),
        mesh=mesh,
        scratch_types=[
            pltpu.VMEM((blks_per_w, blk), jnp.int32),
            *([pltpu.VMEM((HALF, blk, d), jnp.float32)] * NBUF),
            *([pltpu.SemaphoreType.DMA] * NBUF),
            *([pltpu.SemaphoreType.DMA] * NBUF),
        ],
        compiler_params=pltpu.CompilerParams(use_tc_tiling_on_sc=False),
    )
    def k(idx_hbm, w_hbm, out_hbm, idx_v, *bufs_and_sems):
        bufs = bufs_and_sems[:NBUF]
        sem_g = bufs_and_sems[NBUF:2 * NBUF]
        sem_o = bufs_and_sems[2 * NBUF:3 * NBUF]

        wid = lax.axis_index("s") * NC + lax.axis_index("c")
        blk0 = wid * blks_per_w
        pltpu.sync_copy(idx_hbm.at[pl.ds(blk0, blks_per_w)], idx_v)

        def fire_half(f, b):
            for i in range(HALF):
                pltpu.async_copy(
                    w_hbm.at[idx_v.at[f * HALF + i]], bufs[b].at[i], sem_g[b])

        def drain_half(b):
            pltpu.make_async_copy(
                out_hbm.at[pl.ds(0, HALF)], bufs[b], sem_g[b]).wait()

        def start_out(f, b):
            pltpu.async_copy(
                bufs[b], out_hbm.at[pl.ds(blk0 + f * HALF, HALF)], sem_o[b])

        def wait_out(b):
            pltpu.make_async_copy(
                bufs[b], out_hbm.at[pl.ds(0, HALF)], sem_o[b]).wait()

        for b in range(NBUF):
            fire_half(b, b)

        def group_body(g, carry):
            f0 = g * NBUF
            for b in range(NBUF):
                drain_half(b)
                start_out(f0 + b, b)
            for b in range(NBUF):
                wait_out(b)
                fire_half(f0 + NBUF + b, b)
            return carry

        lax.fori_loop(0, ngroups - 1, group_body, 0)

        f0 = (ngroups - 1) * NBUF
        for b in range(NBUF):
            drain_half(b)
            start_out(f0 + b, b)
        for b in range(NBUF):
            wait_out(b)

    return k(idx2d, weight)


def kernel(x, weight):
    b0, b1 = x.shape
    d = weight.shape[1]
    idx2d = x.reshape(-1, BLK).astype(jnp.int32)
    out = _sc_embedding_lookup(idx2d, weight)
    return out.reshape(b0, b1, d)


# speedup vs baseline: 1.4583x; 1.1170x over previous
import functools

import jax
import jax.numpy as jnp
from jax import lax
from jax.experimental import pallas as pl
from jax.experimental.pallas import tpu as pltpu
from jax.experimental.pallas import tpu_sc as plsc

NC = 2
NS = 16
NW = NC * NS
BLK = 128
NBUF = 4
L = 16


def _sc_embedding_lookup(xt, weight, n_pos, batch):
    n_rows, d = weight.shape
    bpw = batch // NW
    ncs = bpw // BLK
    nblocks = n_pos * ncs
    dg, di = d // 8, 8

    mesh = plsc.VectorSubcoreMesh(core_axis_name="c", subcore_axis_name="s")

    @functools.partial(
        pl.kernel,
        out_type=jax.ShapeDtypeStruct((n_pos, dg, batch // BLK, di, BLK),
                                      jnp.float32),
        mesh=mesh,
        scratch_types=[
            pltpu.VMEM((n_pos, bpw), jnp.int32),
            *([pltpu.VMEM((BLK, d), jnp.float32)] * NBUF),
            *([pltpu.VMEM((dg, di, BLK), jnp.float32)] * NBUF),
            *([pltpu.SemaphoreType.DMA] * NBUF),
            *([pltpu.SemaphoreType.DMA] * NBUF),
        ],
        compiler_params=pltpu.CompilerParams(
            use_tc_tiling_on_sc=False, needs_layout_passes=False),
    )
    def k(xt_hbm, w_hbm, out_hbm, xt_v, *bufs_and_sems):
        g = bufs_and_sems[:NBUF]
        t = bufs_and_sems[NBUF:2 * NBUF]
        sem_g = bufs_and_sems[2 * NBUF:3 * NBUF]
        sem_o = bufs_and_sems[3 * NBUF:4 * NBUF]

        wid = lax.axis_index("s") * NC + lax.axis_index("c")
        c0 = wid * bpw
        pltpu.sync_copy(xt_hbm.at[:, pl.ds(c0, bpw)], xt_v)

        jvecs = [lax.iota(jnp.int32, L) + j0 * L for j0 in range(BLK // L)]

        def start_gather(blkid, b):
            s = blkid // ncs
            cs = blkid % ncs
            pltpu.async_copy(
                w_hbm.at[xt_v.at[s, pl.ds(cs * BLK, BLK)]], g[b], sem_g[b])

        def wait_gather(b):
            pltpu.make_async_copy(
                w_hbm.at[xt_v.at[0, pl.ds(0, BLK)]], g[b], sem_g[b]).wait()

        def transpose_block(b):
            for dd in range(d):
                dsplat = jnp.full((L,), dd, jnp.int32)
                for j0 in range(BLK // L):
                    vals = plsc.load_gather(g[b], [jvecs[j0], dsplat])
                    t[b][dd // di, dd % di, pl.ds(j0 * L, L)] = vals

        def out_slice(blkid):
            s = blkid // ncs
            cs = blkid % ncs
            return out_hbm.at[s, :, (c0 // BLK) + cs]

        def start_out(blkid, b):
            pltpu.async_copy(t[b], out_slice(blkid), sem_o[b])

        def wait_out(b):
            pltpu.make_async_copy(t[b], out_hbm.at[0, :, 0], sem_o[b]).wait()

        for b in range(NBUF):
            start_gather(b, b)

        ngroups = nblocks // NBUF

        def group_body(grp, carry):
            b0 = grp * NBUF
            for b in range(NBUF):
                wait_gather(b)
                transpose_block(b)
                start_out(b0 + b, b)
            for b in range(NBUF):
                wait_out(b)
                start_gather(b0 + NBUF + b, b)
            return carry

        lax.fori_loop(0, ngroups - 1, group_body, 0)

        b0 = (ngroups - 1) * NBUF
        for b in range(NBUF):
            wait_gather(b)
            transpose_block(b)
            start_out(b0 + b, b)
        for b in range(NBUF):
            wait_out(b)

    return k(xt, weight)


def kernel(x, weight):
    batch, n_pos = x.shape
    d = weight.shape[1]
    xt = jnp.transpose(x).astype(jnp.int32)
    out5 = _sc_embedding_lookup(xt, weight, n_pos, batch)
    out = jnp.transpose(out5, (2, 4, 0, 1, 3)).reshape(batch, n_pos, d)
    return out


# speedup vs baseline: 1.5994x; 1.0968x over previous
import functools

import jax
import jax.numpy as jnp
from jax import lax
from jax.experimental import pallas as pl
from jax.experimental.pallas import tpu as pltpu
from jax.experimental.pallas import tpu_sc as plsc

NC = 2
NS = 16
NW = NC * NS
BLK = 128
NBUF = 8
L = 16


def _sc_embedding_lookup(xt, weight, n_pos, batch):
    n_rows, d = weight.shape
    bpw = batch // NW
    ncs = bpw // BLK
    nblocks = n_pos * ncs
    dg, di = d // 8, 8

    mesh = plsc.VectorSubcoreMesh(core_axis_name="c", subcore_axis_name="s")

    @functools.partial(
        pl.kernel,
        out_type=jax.ShapeDtypeStruct((n_pos, dg, batch // BLK, di, BLK),
                                      jnp.float32),
        mesh=mesh,
        scratch_types=[
            pltpu.VMEM((n_pos, bpw), jnp.int32),
            *([pltpu.VMEM((BLK, d), jnp.float32)] * NBUF),
            *([pltpu.VMEM((dg, di, BLK), jnp.float32)] * NBUF),
            *([pltpu.SemaphoreType.DMA] * NBUF),
            *([pltpu.SemaphoreType.DMA] * NBUF),
        ],
        compiler_params=pltpu.CompilerParams(
            use_tc_tiling_on_sc=False, needs_layout_passes=False),
    )
    def k(xt_hbm, w_hbm, out_hbm, xt_v, *bufs_and_sems):
        g = bufs_and_sems[:NBUF]
        t = bufs_and_sems[NBUF:2 * NBUF]
        sem_g = bufs_and_sems[2 * NBUF:3 * NBUF]
        sem_o = bufs_and_sems[3 * NBUF:4 * NBUF]

        wid = lax.axis_index("s") * NC + lax.axis_index("c")
        c0 = wid * bpw
        pltpu.sync_copy(xt_hbm.at[:, pl.ds(c0, bpw)], xt_v)

        jvecs = [lax.iota(jnp.int32, L) + j0 * L for j0 in range(BLK // L)]

        def start_gather(blkid, b):
            s = blkid // ncs
            cs = blkid % ncs
            pltpu.async_copy(
                w_hbm.at[xt_v.at[s, pl.ds(cs * BLK, BLK)]], g[b], sem_g[b])

        def wait_gather(b):
            pltpu.make_async_copy(
                w_hbm.at[xt_v.at[0, pl.ds(0, BLK)]], g[b], sem_g[b]).wait()

        def transpose_block(b):
            def dd_body(dd, carry):
                dsplat = jnp.full((L,), 0, jnp.int32) + dd
                dgi = dd // di
                dii = dd % di
                for j0 in range(BLK // L):
                    vals = plsc.load_gather(g[b], [jvecs[j0], dsplat])
                    t[b][dgi, dii, pl.ds(j0 * L, L)] = vals
                return carry

            lax.fori_loop(0, d, dd_body, 0)

        def out_slice(blkid):
            s = blkid // ncs
            cs = blkid % ncs
            return out_hbm.at[s, :, (c0 // BLK) + cs]

        def start_out(blkid, b):
            pltpu.async_copy(t[b], out_slice(blkid), sem_o[b])

        def wait_out(b):
            pltpu.make_async_copy(t[b], out_hbm.at[0, :, 0], sem_o[b]).wait()

        for b in range(NBUF):
            start_gather(b, b)

        ngroups = nblocks // NBUF

        def group_body(grp, carry):
            b0 = grp * NBUF
            for b in range(NBUF):
                wait_gather(b)
                transpose_block(b)
                start_out(b0 + b, b)
            for b in range(NBUF):
                wait_out(b)
                start_gather(lax.rem(b0 + NBUF + b, nblocks), b)
            return carry

        lax.fori_loop(0, ngroups, group_body, 0)

        for b in range(NBUF):
            wait_gather(b)

    return k(xt, weight)


def kernel(x, weight):
    batch, n_pos = x.shape
    d = weight.shape[1]
    xt = jnp.transpose(x).astype(jnp.int32)
    out5 = _sc_embedding_lookup(xt, weight, n_pos, batch)
    out = jnp.transpose(out5, (2, 4, 0, 1, 3)).reshape(batch, n_pos, d)
    return out
